# baseline (speedup 1.0000x reference)
import functools

import jax
import jax.numpy as jnp
from jax import lax
from jax.experimental import pallas as pl
from jax.experimental.pallas import tpu as pltpu
from jax.experimental.pallas import tpu_sc as plsc

_N = 10000
_T = 10
_E = 320000
_VOCAB = 100001
_H = 128
_OUT = 16

_NC = 2
_NS = 16
_NW = _NC * _NS
_CK = 128

_EMB_CHUNKS = 25
_EMB_PER_W = _EMB_CHUNKS * _CK
_BP = _NW * _EMB_PER_W

_G_CHUNKS = 81
_G_PER_W = _G_CHUNKS * _CK
_EP = _NW * _G_PER_W

_NP = 10240
_ROWS_PER_TILE = _NP // _NS

_HC = 64
_HCN = _G_PER_W // _HC

_NB = 512
_GRID = _NP // _NB


@functools.cache
def _sc_mesh():
    return plsc.VectorSubcoreMesh(core_axis_name="c", subcore_axis_name="s",
                                  num_cores=_NC, num_subcores=_NS)


def _emb_body(emb_hbm, idx_hbm, out_hbm, idx_sl, rows0, rows1, sem0, sem1):
    c = lax.axis_index("c")
    s = lax.axis_index("s")
    wid = s * _NC + c
    base = wid * _EMB_CHUNKS

    pltpu.sync_copy(idx_hbm.at[wid], idx_sl)
    pltpu.async_copy(emb_hbm.at[idx_sl.at[0]], rows0, sem0)

    def chunk(j, cur, cur_sem, nxt, nxt_sem):
        pltpu.make_async_copy(emb_hbm.at[idx_sl.at[j]], cur, cur_sem).wait()
        jn = jnp.minimum(j + 1, _EMB_CHUNKS - 1)
        pltpu.async_copy(emb_hbm.at[idx_sl.at[jn]], nxt, nxt_sem)
        pltpu.sync_copy(cur, out_hbm.at[pl.ds((base + j) * _CK, _CK)])

    @pl.loop(0, _EMB_CHUNKS // 2)
    def _pair(p):
        chunk(2 * p, rows0, sem0, rows1, sem1)
        chunk(2 * p + 1, rows1, sem1, rows0, sem0)

    chunk(_EMB_CHUNKS - 1, rows0, sem0, rows1, sem1)
    pltpu.make_async_copy(emb_hbm.at[idx_sl.at[_EMB_CHUNKS - 1]],
                          rows1, sem1).wait()


@functools.cache
def _emb_kernel():
    return pl.kernel(
        _emb_body,
        out_type=jax.ShapeDtypeStruct((_BP, _H), jnp.float32),
        mesh=_sc_mesh(),
        scratch_types=[
            pltpu.VMEM((_EMB_CHUNKS, _CK), jnp.int32),
            pltpu.VMEM((_CK, _H), jnp.float32),
            pltpu.VMEM((_CK, _H), jnp.float32),
            pltpu.SemaphoreType.DMA,
            pltpu.SemaphoreType.DMA,
        ],
    )


def _emb_call(emb, idx2d):
    return _emb_kernel()(emb, idx2d)


def _deg_body(dst_hbm, w_hbm, out_hbm, dst_sl, w_sl, z_v, deg_sh):
    c = lax.axis_index("c")
    s = lax.axis_index("s")
    wid = s * _NC + c

    pltpu.sync_copy(dst_hbm.at[wid], dst_sl)
    pltpu.sync_copy(w_hbm.at[wid], w_sl)

    @pl.loop(0, _ROWS_PER_TILE // 16)
    def _zfill(j):
        z_v[pl.ds(j * 16, 16)] = jnp.zeros((16,), jnp.float32)

    pltpu.sync_copy(z_v, deg_sh.at[pl.ds(s * _ROWS_PER_TILE, _ROWS_PER_TILE)])
    plsc.subcore_barrier()

    @pl.loop(0, _G_CHUNKS)
    def _chunk(k):
        pltpu.sync_copy(w_sl.at[k], deg_sh.at[dst_sl.at[k]], add=True)

    plsc.subcore_barrier()
    r0 = s * _ROWS_PER_TILE
    pltpu.sync_copy(deg_sh.at[pl.ds(r0, _ROWS_PER_TILE)],
                    out_hbm.at[c, pl.ds(r0, _ROWS_PER_TILE)])


@functools.cache
def _deg_kernel():
    return pl.kernel(
        _deg_body,
        out_type=jax.ShapeDtypeStruct((_NC, _NP), jnp.float32),
        mesh=_sc_mesh(),
        scratch_types=[
            pltpu.VMEM((_G_CHUNKS, _CK), jnp.int32),
            pltpu.VMEM((_G_CHUNKS, _CK), jnp.float32),
            pltpu.VMEM((_ROWS_PER_TILE,), jnp.float32),
            pltpu.VMEM_SHARED((_NP,), jnp.float32),
        ],
    )


def _deg_call(dst2d, w2d):
    return _deg_kernel()(dst2d, w2d)


def _agg_body(hs_hbm, src_hbm, dst_hbm, w_hbm, out_hbm,
              src_sl, dst_sl, w_v, dstb, rows0, agg_sh, sem0, sem1, semw):
    c = lax.axis_index("c")
    s = lax.axis_index("s")
    wid = s * _NC + c

    pltpu.sync_copy(src_hbm.at[wid], src_sl)
    pltpu.sync_copy(dst_hbm.at[wid], dst_sl)

    @pl.loop(0, _CK)
    def _zrow(i):
        for q in range(_H // 16):
            rows0[i, pl.ds(q * 16, 16)] = jnp.zeros((16,), jnp.float32)

    @pl.loop(0, _ROWS_PER_TILE // _CK)
    def _zero(j):
        pltpu.sync_copy(rows0, agg_sh.at[pl.ds(s * _ROWS_PER_TILE + j * _CK, _CK)])

    plsc.subcore_barrier()

    pltpu.async_copy(w_hbm.at[wid, 0], w_v, semw)
    pltpu.async_copy(hs_hbm.at[src_sl.at[0, pl.ds(0, _HC)]],
                     rows0.at[pl.ds(0, _HC)], sem0)

    def _half(k, p, r0, cur_sem, kn, pn, r0n, nxt_sem):
        pltpu.make_async_copy(hs_hbm.at[src_sl.at[k, pl.ds(p * _HC, _HC)]],
                              rows0.at[pl.ds(r0, _HC)], cur_sem).wait()
        pltpu.async_copy(hs_hbm.at[src_sl.at[kn, pl.ds(pn * _HC, _HC)]],
                         rows0.at[pl.ds(r0n, _HC)], nxt_sem)
        for q in range(_HC // 16):
            dstb[pl.ds(q * 16, 16)] = dst_sl[k, pl.ds(p * _HC + q * 16, 16)]
        @pl.loop(0, _HC // 16, unroll=2)
        def _scale(g):
            wv16 = w_v[0, pl.ds(p * _HC + g * 16, 16)]
            for e in range(16):
                wvec = jnp.full((16,), wv16[e], jnp.float32)
                i = r0 + g * 16 + e
                for q in range(_H // 16):
                    rows0[i, pl.ds(q * 16, 16)] = rows0[i, pl.ds(q * 16, 16)] * wvec

        pltpu.sync_copy(rows0.at[pl.ds(r0, _HC)], agg_sh.at[dstb], add=True)

    @pl.loop(0, _G_CHUNKS)
    def _chunk(k):
        pltpu.make_async_copy(w_hbm.at[wid, k], w_v, semw).wait()
        kn = jnp.minimum(k + 1, _G_CHUNKS - 1)
        _half(k, 0, 0, sem0, k, 1, _HC, sem1)
        _half(k, 1, _HC, sem1, kn, 0, 0, sem0)
        pltpu.async_copy(w_hbm.at[wid, kn], w_v, semw)

    pltpu.make_async_copy(w_hbm.at[wid, _G_CHUNKS - 1], w_v, semw).wait()
    pltpu.make_async_copy(hs_hbm.at[src_sl.at[0, pl.ds(0, _HC)]],
                          rows0.at[pl.ds(0, _HC)], sem0).wait()

    plsc.subcore_barrier()

    @pl.loop(0, _ROWS_PER_TILE // _CK)
    def _out(j):
        r0 = s * _ROWS_PER_TILE + j * _CK
        pltpu.sync_copy(agg_sh.at[pl.ds(r0, _CK)], out_hbm.at[c, pl.ds(r0, _CK)])


@functools.cache
def _agg_kernel():
    return pl.kernel(
        _agg_body,
        out_type=jax.ShapeDtypeStruct((_NC, _NP, _H), jnp.float32),
        mesh=_sc_mesh(),
        scratch_types=[
            pltpu.VMEM((_G_CHUNKS, _CK), jnp.int32),
            pltpu.VMEM((_G_CHUNKS, _CK), jnp.int32),
            pltpu.VMEM((1, _CK), jnp.float32),
            pltpu.VMEM((_HC,), jnp.int32),
            pltpu.VMEM((_CK, _H), jnp.float32),
            pltpu.VMEM_SHARED((_NP, _H), jnp.float32),
            pltpu.SemaphoreType.DMA,
            pltpu.SemaphoreType.DMA,
            pltpu.SemaphoreType.DMA,
        ],
    )


def _agg_call(hs, src2d, dst2d, w2d):
    return _agg_kernel()(hs, src2d, dst2d,
                         w2d.reshape(_NW, _G_CHUNKS, 1, _CK))


def _sig(x):
    return 0.5 * jnp.tanh(0.5 * x) + 0.5


def _lstm_cell(g, c_prev):
    ig = _sig(g[:, 0:_H])
    fg = _sig(g[:, _H:2 * _H])
    gg = jnp.tanh(g[:, 2 * _H:3 * _H])
    og = _sig(g[:, 3 * _H:4 * _H])
    c = fg * c_prev + ig * gg
    return og * jnp.tanh(c), c


def _dot(a, b):
    return jnp.dot(a, b, preferred_element_type=jnp.float32)


def _lstm_body(xe_ref, oh_ref, degp_ref,
               w0f_ref, b0f_ref, w0r_ref, b0r_ref,
               w1f_ref, b1f_ref, w1r_ref, b1r_ref,
               w1_ref, out_ref, hcat):
    xe = xe_ref[...].reshape(_NB, _T, _H)

    h = jnp.zeros((_NB, _H), jnp.float32)
    cst = h
    w, b = w0f_ref[...], b0f_ref[...]
    for t in range(_T):
        g = _dot(jnp.concatenate([xe[:, t, :], h], axis=1), w) + b
        h, cst = _lstm_cell(g, cst)
        hcat[:, t, 0:_H] = h

    h = jnp.zeros((_NB, _H), jnp.float32)
    cst = h
    w, b = w0r_ref[...], b0r_ref[...]
    for t in reversed(range(_T)):
        g = _dot(jnp.concatenate([xe[:, t, :], h], axis=1), w) + b
        h, cst = _lstm_cell(g, cst)
        hcat[:, t, _H:2 * _H] = h

    oh = oh_ref[...]

    h = jnp.zeros((_NB, _H), jnp.float32)
    cst = h
    lastf = jnp.zeros((_NB, _H), jnp.float32)
    w, b = w1f_ref[...], b1f_ref[...]
    for t in range(_T):
        g = _dot(jnp.concatenate([hcat[:, t, :], h], axis=1), w) + b
        h, cst = _lstm_cell(g, cst)
        lastf = lastf + oh[:, t:t + 1] * h

    h = jnp.zeros((_NB, _H), jnp.float32)
    cst = h
    lastr = jnp.zeros((_NB, _H), jnp.float32)
    w, b = w1r_ref[...], b1r_ref[...]
    for t in reversed(range(_T)):
        g = _dot(jnp.concatenate([hcat[:, t, :], h], axis=1), w) + b
        h, cst = _lstm_cell(g, cst)
        lastr = lastr + oh[:, t:t + 1] * h

    degp = degp_ref[...]
    deg = degp[0] + degp[1]
    dis = jnp.where(deg > 0, jax.lax.rsqrt(deg), 0.0)

    hs = _dot(jnp.concatenate([lastf, lastr], axis=1), w1_ref[...])
    out_ref[...] = hs * dis[:, None]


def _lstm_call(xe_full, onehot, deg_parts, ws):
    full = lambda shape: pl.BlockSpec(shape, lambda i: (0,) * len(shape))
    wspecs = []
    for wmat in ws:
        wspecs.append(full(tuple(wmat.shape)))
    return pl.pallas_call(
        _lstm_body,
        grid=(_GRID,),
        in_specs=[
            pl.BlockSpec((_NB * _T, _H), lambda i: (i, 0)),
            pl.BlockSpec((_NB, _T), lambda i: (i, 0)),
            pl.BlockSpec((_NC, _NB), lambda i: (0, i)),
        ] + wspecs,
        out_specs=pl.BlockSpec((_NB, _H), lambda i: (i, 0)),
        out_shape=jax.ShapeDtypeStruct((_NP, _H), jnp.float32),
        scratch_shapes=[
            pltpu.VMEM((_NB, _T, 2 * _H), jnp.float32),
        ],
    )(xe_full, onehot, deg_parts, *ws)


def _dis_from(degp_ref):
    degp = degp_ref[...]
    deg = degp[0] + degp[1]
    return jnp.where(deg > 0, jax.lax.rsqrt(deg), 0.0)


def _mid_body(sp_ref, degp_ref, b1_ref, w2_ref, out_ref):
    sp = sp_ref[...]
    dis = _dis_from(degp_ref)
    g = jnp.maximum(dis[:, None] * (sp[0] + sp[1]) + b1_ref[...], 0.0)
    out_ref[...] = _dot(g, w2_ref[...]) * dis[:, None]


def _mid_call(s_parts, deg_parts, b1, w2):
    return pl.pallas_call(
        _mid_body,
        grid=(_GRID,),
        in_specs=[
            pl.BlockSpec((_NC, _NB, _H), lambda i: (0, i, 0)),
            pl.BlockSpec((_NC, _NB), lambda i: (0, i)),
            pl.BlockSpec((1, _H), lambda i: (0, 0)),
            pl.BlockSpec((_H, _H), lambda i: (0, 0)),
        ],
        out_specs=pl.BlockSpec((_NB, _H), lambda i: (i, 0)),
        out_shape=jax.ShapeDtypeStruct((_NP, _H), jnp.float32),
    )(s_parts, deg_parts, b1, w2)


def _out_body(sp_ref, degp_ref, b2_ref, fcw_ref, fcb_ref, out_ref):
    sp = sp_ref[...]
    dis = _dis_from(degp_ref)
    g = jnp.maximum(dis[:, None] * (sp[0] + sp[1]) + b2_ref[...], 0.0)
    out_ref[...] = _dot(g, fcw_ref[...]) + fcb_ref[...]


def _final_call(s_parts, deg_parts, b2, fcw, fcb):
    return pl.pallas_call(
        _out_body,
        grid=(_GRID,),
        in_specs=[
            pl.BlockSpec((_NC, _NB, _H), lambda i: (0, i, 0)),
            pl.BlockSpec((_NC, _NB), lambda i: (0, i)),
            pl.BlockSpec((1, _H), lambda i: (0, 0)),
            pl.BlockSpec((_H, _OUT), lambda i: (0, 0)),
            pl.BlockSpec((1, _OUT), lambda i: (0, 0)),
        ],
        out_specs=pl.BlockSpec((_NB, _OUT), lambda i: (i, 0)),
        out_shape=jax.ShapeDtypeStruct((_NP, _OUT), jnp.float32),
    )(s_parts, deg_parts, b2, fcw, fcb)


def kernel(x, lengths, edge_index, edge_weight, emb,
           Wih_l0_f, Whh_l0_f, bih_l0_f, bhh_l0_f,
           Wih_l0_r, Whh_l0_r, bih_l0_r, bhh_l0_r,
           Wih_l1_f, Whh_l1_f, bih_l1_f, bhh_l1_f,
           Wih_l1_r, Whh_l1_r, bih_l1_r, bhh_l1_r,
           W1, b1, W2, b2, fcW, fcb):
    xf = x.reshape(-1).astype(jnp.int32)
    pad_i = (jnp.arange(_BP - _N * _T, dtype=jnp.int32) * 977) % _VOCAB
    idx2d = jnp.concatenate([xf, pad_i]).reshape(_NW, _EMB_CHUNKS, _CK)
    xe_full = _emb_call(emb, idx2d)

    ar = jnp.arange(_N, dtype=jnp.int32)
    npad = _EP - (_E + _N)
    pad_n = (jnp.arange(npad, dtype=jnp.int32) * 131) % _N
    src2d = jnp.concatenate([edge_index[0].astype(jnp.int32), ar, pad_n]
                            ).reshape(_NW, _G_CHUNKS, _CK)
    dst2d = jnp.concatenate([edge_index[1].astype(jnp.int32), ar, pad_n]
                            ).reshape(_NW, _G_CHUNKS, _CK)
    w2d = jnp.concatenate([edge_weight.astype(jnp.float32),
                           jnp.ones((_N,), jnp.float32),
                           jnp.zeros((npad,), jnp.float32)]
                          ).reshape(_NW, _G_CHUNKS, _CK)

    deg_parts = _deg_call(dst2d, w2d)

    lengths_p = jnp.concatenate([lengths.astype(jnp.int32),
                                 jnp.ones((_NP - _N,), jnp.int32)])
    onehot = (jnp.arange(_T, dtype=jnp.int32)[None, :]
              == (lengths_p - 1)[:, None]).astype(jnp.float32)
    cat = lambda a, b: jnp.concatenate([a.T, b.T], axis=0)
    hs1 = _lstm_call(
        xe_full, onehot, deg_parts,
        (cat(Wih_l0_f, Whh_l0_f), (bih_l0_f + bhh_l0_f).reshape(1, -1),
         cat(Wih_l0_r, Whh_l0_r), (bih_l0_r + bhh_l0_r).reshape(1, -1),
         cat(Wih_l1_f, Whh_l1_f), (bih_l1_f + bhh_l1_f).reshape(1, -1),
         cat(Wih_l1_r, Whh_l1_r), (bih_l1_r + bhh_l1_r).reshape(1, -1),
         W1))

    s1 = _agg_call(hs1, src2d, dst2d, w2d)
    hs2 = _mid_call(s1, deg_parts, b1.reshape(1, -1), W2)

    s2 = _agg_call(hs2, src2d, dst2d, w2d)
    out = _final_call(s2, deg_parts, b2.reshape(1, -1), fcW, fcb.reshape(1, -1))
    return out[:_N]


# speedup vs baseline: 1.0029x; 1.0029x over previous
import functools

import jax
import jax.numpy as jnp
from jax import lax
from jax.experimental import pallas as pl
from jax.experimental.pallas import tpu as pltpu
from jax.experimental.pallas import tpu_sc as plsc

_N = 10000
_T = 10
_E = 320000
_VOCAB = 100001
_H = 128
_OUT = 16

_NC = 2
_NS = 16
_NW = _NC * _NS
_CK = 128

_EMB_CHUNKS = 25
_EMB_PER_W = _EMB_CHUNKS * _CK
_BP = _NW * _EMB_PER_W

_G_CHUNKS = 81
_G_PER_W = _G_CHUNKS * _CK
_EP = _NW * _G_PER_W

_NP = 10240
_ROWS_PER_TILE = _NP // _NS

_HC = 64
_HCN = _G_PER_W // _HC

_NB = 512
_GRID = _NP // _NB


@functools.cache
def _sc_mesh():
    return plsc.VectorSubcoreMesh(core_axis_name="c", subcore_axis_name="s",
                                  num_cores=_NC, num_subcores=_NS)


def _emb_body(emb_hbm, idx_hbm, out_hbm, idx_sl, rows0, rows1, sem0, sem1):
    c = lax.axis_index("c")
    s = lax.axis_index("s")
    wid = s * _NC + c
    base = wid * _EMB_CHUNKS

    pltpu.sync_copy(idx_hbm.at[wid], idx_sl)
    pltpu.async_copy(emb_hbm.at[idx_sl.at[0]], rows0, sem0)

    def chunk(j, cur, cur_sem, nxt, nxt_sem):
        pltpu.make_async_copy(emb_hbm.at[idx_sl.at[j]], cur, cur_sem).wait()
        jn = jnp.minimum(j + 1, _EMB_CHUNKS - 1)
        pltpu.async_copy(emb_hbm.at[idx_sl.at[jn]], nxt, nxt_sem)
        pltpu.sync_copy(cur, out_hbm.at[pl.ds((base + j) * _CK, _CK)])

    @pl.loop(0, _EMB_CHUNKS // 2)
    def _pair(p):
        chunk(2 * p, rows0, sem0, rows1, sem1)
        chunk(2 * p + 1, rows1, sem1, rows0, sem0)

    chunk(_EMB_CHUNKS - 1, rows0, sem0, rows1, sem1)
    pltpu.make_async_copy(emb_hbm.at[idx_sl.at[_EMB_CHUNKS - 1]],
                          rows1, sem1).wait()


@functools.cache
def _emb_kernel():
    return pl.kernel(
        _emb_body,
        out_type=jax.ShapeDtypeStruct((_BP, _H), jnp.float32),
        mesh=_sc_mesh(),
        scratch_types=[
            pltpu.VMEM((_EMB_CHUNKS, _CK), jnp.int32),
            pltpu.VMEM((_CK, _H), jnp.float32),
            pltpu.VMEM((_CK, _H), jnp.float32),
            pltpu.SemaphoreType.DMA,
            pltpu.SemaphoreType.DMA,
        ],
    )


def _emb_call(emb, idx2d):
    return _emb_kernel()(emb, idx2d)


def _deg_body(dst_hbm, w_hbm, out_hbm, dst_sl, w_sl, z_v, deg_sh):
    c = lax.axis_index("c")
    s = lax.axis_index("s")
    wid = s * _NC + c

    pltpu.sync_copy(dst_hbm.at[wid], dst_sl)
    pltpu.sync_copy(w_hbm.at[wid], w_sl)

    @pl.loop(0, _ROWS_PER_TILE // 16)
    def _zfill(j):
        z_v[pl.ds(j * 16, 16)] = jnp.zeros((16,), jnp.float32)

    pltpu.sync_copy(z_v, deg_sh.at[pl.ds(s * _ROWS_PER_TILE, _ROWS_PER_TILE)])
    plsc.subcore_barrier()

    @pl.loop(0, _G_CHUNKS)
    def _chunk(k):
        pltpu.sync_copy(w_sl.at[k], deg_sh.at[dst_sl.at[k]], add=True)

    plsc.subcore_barrier()
    r0 = s * _ROWS_PER_TILE
    pltpu.sync_copy(deg_sh.at[pl.ds(r0, _ROWS_PER_TILE)],
                    out_hbm.at[c, pl.ds(r0, _ROWS_PER_TILE)])


@functools.cache
def _deg_kernel():
    return pl.kernel(
        _deg_body,
        out_type=jax.ShapeDtypeStruct((_NC, _NP), jnp.float32),
        mesh=_sc_mesh(),
        scratch_types=[
            pltpu.VMEM((_G_CHUNKS, _CK), jnp.int32),
            pltpu.VMEM((_G_CHUNKS, _CK), jnp.float32),
            pltpu.VMEM((_ROWS_PER_TILE,), jnp.float32),
            pltpu.VMEM_SHARED((_NP,), jnp.float32),
        ],
    )


def _deg_call(dst2d, w2d):
    return _deg_kernel()(dst2d, w2d)


def _agg_body(hs_hbm, src_hbm, dst_hbm, w_hbm, out_hbm,
              src_sl, dst_sl, w_v, dstb, rows0, agg_sh, sem0, sem1, semw):
    c = lax.axis_index("c")
    s = lax.axis_index("s")
    wid = s * _NC + c

    pltpu.sync_copy(src_hbm.at[wid], src_sl)
    pltpu.sync_copy(dst_hbm.at[wid], dst_sl)

    @pl.loop(0, _CK)
    def _zrow(i):
        for q in range(_H // 16):
            rows0[i, pl.ds(q * 16, 16)] = jnp.zeros((16,), jnp.float32)

    @pl.loop(0, _ROWS_PER_TILE // _CK)
    def _zero(j):
        pltpu.sync_copy(rows0, agg_sh.at[pl.ds(s * _ROWS_PER_TILE + j * _CK, _CK)])

    plsc.subcore_barrier()

    pltpu.async_copy(w_hbm.at[wid, 0], w_v, semw)
    pltpu.async_copy(hs_hbm.at[src_sl.at[0, pl.ds(0, _HC)]],
                     rows0.at[pl.ds(0, _HC)], sem0)

    def _half(k, p, r0, cur_sem, kn, pn, r0n, nxt_sem):
        pltpu.make_async_copy(hs_hbm.at[src_sl.at[k, pl.ds(p * _HC, _HC)]],
                              rows0.at[pl.ds(r0, _HC)], cur_sem).wait()
        pltpu.async_copy(hs_hbm.at[src_sl.at[kn, pl.ds(pn * _HC, _HC)]],
                         rows0.at[pl.ds(r0n, _HC)], nxt_sem)
        for q in range(_HC // 16):
            dstb[pl.ds(q * 16, 16)] = dst_sl[k, pl.ds(p * _HC + q * 16, 16)]
        @pl.loop(0, _HC // 16)
        def _scale(g):
            wv16 = w_v[0, pl.ds(p * _HC + g * 16, 16)]
            for e in range(16):
                wvec = jnp.full((16,), wv16[e], jnp.float32)
                i = r0 + g * 16 + e
                for q in range(_H // 16):
                    rows0[i, pl.ds(q * 16, 16)] = rows0[i, pl.ds(q * 16, 16)] * wvec

        pltpu.sync_copy(rows0.at[pl.ds(r0, _HC)], agg_sh.at[dstb], add=True)

    @pl.loop(0, _G_CHUNKS)
    def _chunk(k):
        pltpu.make_async_copy(w_hbm.at[wid, k], w_v, semw).wait()
        kn = jnp.minimum(k + 1, _G_CHUNKS - 1)
        _half(k, 0, 0, sem0, k, 1, _HC, sem1)
        _half(k, 1, _HC, sem1, kn, 0, 0, sem0)
        pltpu.async_copy(w_hbm.at[wid, kn], w_v, semw)

    pltpu.make_async_copy(w_hbm.at[wid, _G_CHUNKS - 1], w_v, semw).wait()
    pltpu.make_async_copy(hs_hbm.at[src_sl.at[0, pl.ds(0, _HC)]],
                          rows0.at[pl.ds(0, _HC)], sem0).wait()

    plsc.subcore_barrier()

    @pl.loop(0, _ROWS_PER_TILE // _CK)
    def _out(j):
        r0 = s * _ROWS_PER_TILE + j * _CK
        pltpu.sync_copy(agg_sh.at[pl.ds(r0, _CK)], out_hbm.at[c, pl.ds(r0, _CK)])


@functools.cache
def _agg_kernel():
    return pl.kernel(
        _agg_body,
        out_type=jax.ShapeDtypeStruct((_NC, _NP, _H), jnp.float32),
        mesh=_sc_mesh(),
        scratch_types=[
            pltpu.VMEM((_G_CHUNKS, _CK), jnp.int32),
            pltpu.VMEM((_G_CHUNKS, _CK), jnp.int32),
            pltpu.VMEM((1, _CK), jnp.float32),
            pltpu.VMEM((_HC,), jnp.int32),
            pltpu.VMEM((_CK, _H), jnp.float32),
            pltpu.VMEM_SHARED((_NP, _H), jnp.float32),
            pltpu.SemaphoreType.DMA,
            pltpu.SemaphoreType.DMA,
            pltpu.SemaphoreType.DMA,
        ],
    )


def _agg_call(hs, src2d, dst2d, w2d):
    return _agg_kernel()(hs, src2d, dst2d,
                         w2d.reshape(_NW, _G_CHUNKS, 1, _CK))


def _sig(x):
    return 0.5 * jnp.tanh(0.5 * x) + 0.5


def _lstm_cell(g, c_prev):
    ig = _sig(g[:, 0:_H])
    fg = _sig(g[:, _H:2 * _H])
    gg = jnp.tanh(g[:, 2 * _H:3 * _H])
    og = _sig(g[:, 3 * _H:4 * _H])
    c = fg * c_prev + ig * gg
    return og * jnp.tanh(c), c


def _dot(a, b):
    return jnp.dot(a, b, preferred_element_type=jnp.float32)


def _lstm_body(xe_ref, oh_ref, degp_ref,
               w0f_ref, b0f_ref, w0r_ref, b0r_ref,
               w1f_ref, b1f_ref, w1r_ref, b1r_ref,
               w1_ref, out_ref, hcat):
    xe = xe_ref[...].reshape(_NB, _T, _H)

    h = jnp.zeros((_NB, _H), jnp.float32)
    cst = h
    w, b = w0f_ref[...], b0f_ref[...]
    for t in range(_T):
        g = _dot(jnp.concatenate([xe[:, t, :], h], axis=1), w) + b
        h, cst = _lstm_cell(g, cst)
        hcat[:, t, 0:_H] = h

    h = jnp.zeros((_NB, _H), jnp.float32)
    cst = h
    w, b = w0r_ref[...], b0r_ref[...]
    for t in reversed(range(_T)):
        g = _dot(jnp.concatenate([xe[:, t, :], h], axis=1), w) + b
        h, cst = _lstm_cell(g, cst)
        hcat[:, t, _H:2 * _H] = h

    oh = oh_ref[...]

    h = jnp.zeros((_NB, _H), jnp.float32)
    cst = h
    lastf = jnp.zeros((_NB, _H), jnp.float32)
    w, b = w1f_ref[...], b1f_ref[...]
    for t in range(_T):
        g = _dot(jnp.concatenate([hcat[:, t, :], h], axis=1), w) + b
        h, cst = _lstm_cell(g, cst)
        lastf = lastf + oh[:, t:t + 1] * h

    h = jnp.zeros((_NB, _H), jnp.float32)
    cst = h
    lastr = jnp.zeros((_NB, _H), jnp.float32)
    w, b = w1r_ref[...], b1r_ref[...]
    for t in reversed(range(_T)):
        g = _dot(jnp.concatenate([hcat[:, t, :], h], axis=1), w) + b
        h, cst = _lstm_cell(g, cst)
        lastr = lastr + oh[:, t:t + 1] * h

    degp = degp_ref[...]
    deg = degp[0] + degp[1]
    dis = jnp.where(deg > 0, jax.lax.rsqrt(deg), 0.0)

    hs = _dot(jnp.concatenate([lastf, lastr], axis=1), w1_ref[...])
    out_ref[...] = hs * dis[:, None]


def _lstm_call(xe_full, onehot, deg_parts, ws):
    full = lambda shape: pl.BlockSpec(shape, lambda i: (0,) * len(shape))
    wspecs = []
    for wmat in ws:
        wspecs.append(full(tuple(wmat.shape)))
    return pl.pallas_call(
        _lstm_body,
        grid=(_GRID,),
        in_specs=[
            pl.BlockSpec((_NB * _T, _H), lambda i: (i, 0)),
            pl.BlockSpec((_NB, _T), lambda i: (i, 0)),
            pl.BlockSpec((_NC, _NB), lambda i: (0, i)),
        ] + wspecs,
        out_specs=pl.BlockSpec((_NB, _H), lambda i: (i, 0)),
        out_shape=jax.ShapeDtypeStruct((_NP, _H), jnp.float32),
        scratch_shapes=[
            pltpu.VMEM((_NB, _T, 2 * _H), jnp.float32),
        ],
    )(xe_full, onehot, deg_parts, *ws)


def _dis_from(degp_ref):
    degp = degp_ref[...]
    deg = degp[0] + degp[1]
    return jnp.where(deg > 0, jax.lax.rsqrt(deg), 0.0)


def _mid_body(sp_ref, degp_ref, b1_ref, w2_ref, out_ref):
    sp = sp_ref[...]
    dis = _dis_from(degp_ref)
    g = jnp.maximum(dis[:, None] * (sp[0] + sp[1]) + b1_ref[...], 0.0)
    out_ref[...] = _dot(g, w2_ref[...]) * dis[:, None]


def _mid_call(s_parts, deg_parts, b1, w2):
    return pl.pallas_call(
        _mid_body,
        grid=(_GRID,),
        in_specs=[
            pl.BlockSpec((_NC, _NB, _H), lambda i: (0, i, 0)),
            pl.BlockSpec((_NC, _NB), lambda i: (0, i)),
            pl.BlockSpec((1, _H), lambda i: (0, 0)),
            pl.BlockSpec((_H, _H), lambda i: (0, 0)),
        ],
        out_specs=pl.BlockSpec((_NB, _H), lambda i: (i, 0)),
        out_shape=jax.ShapeDtypeStruct((_NP, _H), jnp.float32),
    )(s_parts, deg_parts, b1, w2)


def _out_body(sp_ref, degp_ref, b2_ref, fcw_ref, fcb_ref, out_ref):
    sp = sp_ref[...]
    dis = _dis_from(degp_ref)
    g = jnp.maximum(dis[:, None] * (sp[0] + sp[1]) + b2_ref[...], 0.0)
    out_ref[...] = _dot(g, fcw_ref[...]) + fcb_ref[...]


def _final_call(s_parts, deg_parts, b2, fcw, fcb):
    return pl.pallas_call(
        _out_body,
        grid=(_GRID,),
        in_specs=[
            pl.BlockSpec((_NC, _NB, _H), lambda i: (0, i, 0)),
            pl.BlockSpec((_NC, _NB), lambda i: (0, i)),
            pl.BlockSpec((1, _H), lambda i: (0, 0)),
            pl.BlockSpec((_H, _OUT), lambda i: (0, 0)),
            pl.BlockSpec((1, _OUT), lambda i: (0, 0)),
        ],
        out_specs=pl.BlockSpec((_NB, _OUT), lambda i: (i, 0)),
        out_shape=jax.ShapeDtypeStruct((_NP, _OUT), jnp.float32),
    )(s_parts, deg_parts, b2, fcw, fcb)


def kernel(x, lengths, edge_index, edge_weight, emb,
           Wih_l0_f, Whh_l0_f, bih_l0_f, bhh_l0_f,
           Wih_l0_r, Whh_l0_r, bih_l0_r, bhh_l0_r,
           Wih_l1_f, Whh_l1_f, bih_l1_f, bhh_l1_f,
           Wih_l1_r, Whh_l1_r, bih_l1_r, bhh_l1_r,
           W1, b1, W2, b2, fcW, fcb):
    xf = x.reshape(-1).astype(jnp.int32)
    pad_i = (jnp.arange(_BP - _N * _T, dtype=jnp.int32) * 977) % _VOCAB
    idx2d = jnp.concatenate([xf, pad_i]).reshape(_NW, _EMB_CHUNKS, _CK)
    xe_full = _emb_call(emb, idx2d)

    ar = jnp.arange(_N, dtype=jnp.int32)
    npad = _EP - (_E + _N)
    pad_n = (jnp.arange(npad, dtype=jnp.int32) * 131) % _N
    src2d = jnp.concatenate([edge_index[0].astype(jnp.int32), ar, pad_n]
                            ).reshape(_NW, _G_CHUNKS, _CK)
    dst2d = jnp.concatenate([edge_index[1].astype(jnp.int32), ar, pad_n]
                            ).reshape(_NW, _G_CHUNKS, _CK)
    w2d = jnp.concatenate([edge_weight.astype(jnp.float32),
                           jnp.ones((_N,), jnp.float32),
                           jnp.zeros((npad,), jnp.float32)]
                          ).reshape(_NW, _G_CHUNKS, _CK)

    deg_parts = _deg_call(dst2d, w2d)

    lengths_p = jnp.concatenate([lengths.astype(jnp.int32),
                                 jnp.ones((_NP - _N,), jnp.int32)])
    onehot = (jnp.arange(_T, dtype=jnp.int32)[None, :]
              == (lengths_p - 1)[:, None]).astype(jnp.float32)
    cat = lambda a, b: jnp.concatenate([a.T, b.T], axis=0)
    hs1 = _lstm_call(
        xe_full, onehot, deg_parts,
        (cat(Wih_l0_f, Whh_l0_f), (bih_l0_f + bhh_l0_f).reshape(1, -1),
         cat(Wih_l0_r, Whh_l0_r), (bih_l0_r + bhh_l0_r).reshape(1, -1),
         cat(Wih_l1_f, Whh_l1_f), (bih_l1_f + bhh_l1_f).reshape(1, -1),
         cat(Wih_l1_r, Whh_l1_r), (bih_l1_r + bhh_l1_r).reshape(1, -1),
         W1))

    s1 = _agg_call(hs1, src2d, dst2d, w2d)
    hs2 = _mid_call(s1, deg_parts, b1.reshape(1, -1), W2)

    s2 = _agg_call(hs2, src2d, dst2d, w2d)
    out = _final_call(s2, deg_parts, b2.reshape(1, -1), fcW, fcb.reshape(1, -1))
    return out[:_N]
